# Initial kernel scaffold; baseline (speedup 1.0000x reference)
import jax
import jax.numpy as jnp
from jax.experimental import pallas as pl


def kernel(x, w, b):
    raise NotImplementedError("write your pallas kernel here")


# speedup vs baseline: 43.7022x; 43.7022x over previous
import jax
import jax.numpy as jnp
from jax.experimental import pallas as pl
from jax.experimental.pallas import tpu as pltpu

_GRID_SHAPE = (6, 64)
_BATCH = 32
_DIM = 512
_NUM_LAYERS = _GRID_SHAPE[0]
_SPATIAL = _GRID_SHAPE[1]
_NUM_CELLS = _NUM_LAYERS * _SPATIAL


def _wavefront_body(x_ref, w_ref, b_ref, out0_ref, out1_ref, d0_ref):
    l = pl.program_id(0)

    @pl.when(l == 0)
    def _seed():
        d0_ref[...] = x_ref[...]

    w00 = w_ref[0, 0, 0, :]
    w01 = w_ref[0, 0, 1, :]
    w10 = w_ref[0, 1, 0, :]
    w11 = w_ref[0, 1, 1, :]
    b0 = b_ref[0, 0, :]
    b1 = b_ref[0, 1, :]

    def step(s, g1_prev):
        d0 = d0_ref[s]
        g0 = jnp.tanh(b0 + d0 * w00 + g1_prev * w01)
        g1 = jnp.tanh(b1 + d0 * w10 + g1_prev * w11)
        out0_ref[s] = g0
        out1_ref[s] = g1
        d0_ref[s] = g0
        return g1

    jax.lax.fori_loop(0, _SPATIAL, step,
                      jnp.zeros((_BATCH, _DIM), dtype=out0_ref.dtype))


def kernel(x, w, b):
    x_t = jnp.transpose(x, (1, 0, 2))
    out0, out1 = pl.pallas_call(
        _wavefront_body,
        grid=(_NUM_LAYERS,),
        in_specs=[
            pl.BlockSpec((_SPATIAL, _BATCH, _DIM), lambda l: (0, 0, 0)),
            pl.BlockSpec((1, 2, 2, _DIM), lambda l: (l, 0, 0, 0)),
            pl.BlockSpec((1, 2, _DIM), lambda l: (l, 0, 0)),
        ],
        out_specs=[
            pl.BlockSpec((_SPATIAL, _BATCH, _DIM), lambda l: (l, 0, 0)),
            pl.BlockSpec((_SPATIAL, _BATCH, _DIM), lambda l: (l, 0, 0)),
        ],
        out_shape=[
            jax.ShapeDtypeStruct((_NUM_CELLS, _BATCH, _DIM), x.dtype),
            jax.ShapeDtypeStruct((_NUM_CELLS, _BATCH, _DIM), x.dtype),
        ],
        scratch_shapes=[pltpu.VMEM((_SPATIAL, _BATCH, _DIM), x.dtype)],
        compiler_params=pltpu.CompilerParams(
            dimension_semantics=("arbitrary",),
        ),
    )(x_t, w, b)
    return (out0, out1)
